# baseline (speedup 1.0000x reference)
import functools

import jax
import jax.numpy as jnp
from jax import lax
from jax.experimental import pallas as pl
from jax.experimental.pallas import tpu as pltpu
from jax.experimental.pallas import tpu_sc as plsc

D = 64
DW = D // 2
NC = 2
NS = 16
NW = NC * NS
C = 112


def _fold_tables(emb_colors, emb_chars, emb_specials, lin_w, lin_b2d):
    def body(ec, eh, es, w, bvec, pc, ph, ps):
        pc[...] = (jnp.dot(ec[...], w[0:D, :],
                           preferred_element_type=jnp.float32)
                   + bvec[...]).astype(jnp.bfloat16)
        ph[...] = jnp.dot(eh[...], w[D:2 * D, :],
                          preferred_element_type=jnp.float32).astype(jnp.bfloat16)
        ps[...] = jnp.dot(es[...], w[2 * D:3 * D, :],
                          preferred_element_type=jnp.float32).astype(jnp.bfloat16)

    return pl.pallas_call(
        body,
        out_shape=(
            jax.ShapeDtypeStruct((16, D), jnp.bfloat16),
            jax.ShapeDtypeStruct((256, D), jnp.bfloat16),
            jax.ShapeDtypeStruct((256, D), jnp.bfloat16),
        ),
    )(emb_colors, emb_chars, emb_specials, lin_w, lin_b2d)


def _pack_pairs(t):
    v = t.shape[0]
    return lax.bitcast_convert_type(
        t.reshape(v, DW, 2), jnp.int32).reshape(v * DW)


def _sc_embed(colors, chars, specials, pc, ph, ps):
    N = colors.shape[0]
    assert N % (NW * C) == 0
    tpw = N // NW
    nch = tpw // C
    assert nch % 2 == 0

    mesh = plsc.VectorSubcoreMesh(core_axis_name="c", subcore_axis_name="s")

    @functools.partial(
        pl.kernel,
        out_type=jax.ShapeDtypeStruct((N * D,), jnp.float32),
        mesh=mesh,
        compiler_params=pltpu.CompilerParams(needs_layout_passes=False),
        scratch_types=[
            pltpu.VMEM((16 * DW,), jnp.int32),
            pltpu.VMEM((256 * DW,), jnp.int32),
            pltpu.VMEM((256 * DW,), jnp.int32),
            pltpu.VMEM((C,), jnp.int32),
            pltpu.VMEM((C,), jnp.int32),
            pltpu.VMEM((C,), jnp.int32),
            pltpu.VMEM((C,), jnp.int32),
            pltpu.VMEM((C,), jnp.int32),
            pltpu.VMEM((C,), jnp.int32),
            pltpu.VMEM((C * D,), jnp.float32),
            pltpu.VMEM((C * D,), jnp.float32),
            pltpu.SemaphoreType.DMA((2,)),
            pltpu.SemaphoreType.DMA((2,)),
        ],
    )
    def k(colors_h, chars_h, specials_h, pc_h, ph_h, ps_h, out_h,
          tabc, tabh, tabs, ic0, ih0, is0, ic1, ih1, is1, ob0, ob1,
          sem_i, sem_o):
        idx_refs = ((ic0, ih0, is0), (ic1, ih1, is1))
        out_bufs = (ob0, ob1)
        wid = lax.axis_index("s") * NC + lax.axis_index("c")
        base0 = wid * tpw

        pltpu.sync_copy(pc_h, tabc)
        pltpu.sync_copy(ph_h, tabh)
        pltpu.sync_copy(ps_h, tabs)

        iota16 = lax.broadcasted_iota(jnp.int32, (16,), 0)
        outstep = iota16 * D
        mask_hi = jnp.full((16,), -65536, dtype=jnp.int32)

        idx_srcs = (colors_h, chars_h, specials_h)

        def start_idx(i, b):
            base = base0 + i * C
            for j, src in enumerate(idx_srcs):
                pltpu.async_copy(src.at[pl.ds(base, C)], idx_refs[b][j],
                                 sem_i.at[b])

        def wait_idx(i, b):
            base = base0 + i * C
            for j, src in enumerate(idx_srcs):
                pltpu.make_async_copy(src.at[pl.ds(base, C)], idx_refs[b][j],
                                      sem_i.at[b]).wait()

        def out_slice(i):
            return out_h.at[pl.ds((base0 + i * C) * D, C * D)]

        start_idx(0, 0)
        start_idx(1, 1)

        def outer(g, carry):
            for b in range(2):
                i = 2 * g + b
                wait_idx(i, b)

                @pl.when(i >= 2)
                def _():
                    pltpu.make_async_copy(out_bufs[b], out_slice(i - 2),
                                          sem_o.at[b]).wait()

                ob_ref = out_bufs[b]

                def group_body(g2, c2):
                    t0 = g2 * 16
                    rcv = idx_refs[b][0][pl.ds(t0, 16)] * DW
                    rhv = idx_refs[b][1][pl.ds(t0, 16)] * DW
                    rsv = idx_refs[b][2][pl.ds(t0, 16)] * DW
                    obase = outstep + t0 * D
                    for kk in range(DW):
                        vc = plsc.load_gather(tabc, [rcv + kk])
                        vh = plsc.load_gather(tabh, [rhv + kk])
                        vs = plsc.load_gather(tabs, [rsv + kk])
                        s = (plsc.bitcast(vc, jnp.bfloat16)
                             + plsc.bitcast(vh, jnp.bfloat16)
                             + plsc.bitcast(vs, jnp.bfloat16))
                        su = plsc.bitcast(s, jnp.int32)
                        lo = plsc.bitcast(lax.shift_left(su, 16), jnp.float32)
                        hi = plsc.bitcast(su & mask_hi, jnp.float32)
                        plsc.store_scatter(ob_ref, [obase + (2 * kk)], lo)
                        plsc.store_scatter(ob_ref, [obase + (2 * kk + 1)], hi)
                    return c2

                lax.fori_loop(0, C // 16, group_body, 0)
                pltpu.async_copy(out_bufs[b], out_slice(i), sem_o.at[b])

                @pl.when(i + 2 < nch)
                def _():
                    start_idx(i + 2, b)
            return carry

        lax.fori_loop(0, nch // 2, outer, 0)
        for b in range(2):
            pltpu.make_async_copy(out_bufs[b], out_slice(nch - 2 + b),
                                  sem_o.at[b]).wait()

    return k(colors, chars, specials, pc, ph, ps)


def kernel(colors, chars, specials, emb_colors, emb_chars, emb_specials,
           lin_w, lin_b):
    B, H, W = colors.shape
    N = B * H * W
    pc, ph, ps = _fold_tables(emb_colors, emb_chars, emb_specials, lin_w,
                              lin_b.reshape(1, D))
    out_flat = _sc_embed(
        colors.reshape(N), chars.reshape(N), specials.reshape(N),
        _pack_pairs(pc), _pack_pairs(ph), _pack_pairs(ps))
    return out_flat.reshape(B, H, W, D)


# speedup vs baseline: 3.1145x; 3.1145x over previous
import functools

import jax
import jax.numpy as jnp
from jax import lax
from jax.experimental import pallas as pl
from jax.experimental.pallas import tpu as pltpu
from jax.experimental.pallas import tpu_sc as plsc

D = 64
DW = D // 2
NC = 2
NS = 16
NW = NC * NS
C = 112


def _fold_tables(emb_colors, emb_chars, emb_specials, lin_w, lin_b2d):
    def body(ec, eh, es, w, bvec, pc, ph, ps):
        pc[...] = (jnp.dot(ec[...], w[0:D, :],
                           preferred_element_type=jnp.float32)
                   + bvec[...]).astype(jnp.bfloat16)
        ph[...] = jnp.dot(eh[...], w[D:2 * D, :],
                          preferred_element_type=jnp.float32).astype(jnp.bfloat16)
        ps[...] = jnp.dot(es[...], w[2 * D:3 * D, :],
                          preferred_element_type=jnp.float32).astype(jnp.bfloat16)

    return pl.pallas_call(
        body,
        out_shape=(
            jax.ShapeDtypeStruct((16, D), jnp.bfloat16),
            jax.ShapeDtypeStruct((256, D), jnp.bfloat16),
            jax.ShapeDtypeStruct((256, D), jnp.bfloat16),
        ),
    )(emb_colors, emb_chars, emb_specials, lin_w, lin_b2d)


def _pack_pairs(t):
    v = t.shape[0]
    tp = t.reshape(v, 2, 2, 16).transpose(0, 1, 3, 2)
    return lax.bitcast_convert_type(tp.reshape(v, DW, 2),
                                    jnp.int32).reshape(v * DW)


def _sc_embed(colors, chars, specials, pc, ph, ps):
    N = colors.shape[0]
    assert N % (NW * C) == 0
    tpw = N // NW
    nch = tpw // C
    assert nch % 2 == 0

    mesh = plsc.VectorSubcoreMesh(core_axis_name="c", subcore_axis_name="s")

    @functools.partial(
        pl.kernel,
        out_type=jax.ShapeDtypeStruct((N * D,), jnp.float32),
        mesh=mesh,
        compiler_params=pltpu.CompilerParams(needs_layout_passes=False),
        scratch_types=[
            pltpu.VMEM((16 * DW,), jnp.int32),
            pltpu.VMEM((256 * DW,), jnp.int32),
            pltpu.VMEM((256 * DW,), jnp.int32),
            pltpu.VMEM((C,), jnp.int32),
            pltpu.VMEM((C,), jnp.int32),
            pltpu.VMEM((C,), jnp.int32),
            pltpu.VMEM((C,), jnp.int32),
            pltpu.VMEM((C,), jnp.int32),
            pltpu.VMEM((C,), jnp.int32),
            pltpu.VMEM((C * D,), jnp.float32),
            pltpu.VMEM((C * D,), jnp.float32),
            pltpu.SemaphoreType.DMA((2,)),
            pltpu.SemaphoreType.DMA((2,)),
        ],
    )
    def k(colors_h, chars_h, specials_h, pc_h, ph_h, ps_h, out_h,
          tabc, tabh, tabs, ic0, ih0, is0, ic1, ih1, is1, ob0, ob1,
          sem_i, sem_o):
        idx_refs = ((ic0, ih0, is0), (ic1, ih1, is1))
        out_bufs = (ob0, ob1)
        wid = lax.axis_index("s") * NC + lax.axis_index("c")
        base0 = wid * tpw

        pltpu.sync_copy(pc_h, tabc)
        pltpu.sync_copy(ph_h, tabh)
        pltpu.sync_copy(ps_h, tabs)

        mask_hi = jnp.full((16,), -65536, dtype=jnp.int32)

        idx_srcs = (colors_h, chars_h, specials_h)

        def start_idx(i, b):
            base = base0 + i * C
            for j, src in enumerate(idx_srcs):
                pltpu.async_copy(src.at[pl.ds(base, C)], idx_refs[b][j],
                                 sem_i.at[b])

        def wait_idx(i, b):
            base = base0 + i * C
            for j, src in enumerate(idx_srcs):
                pltpu.make_async_copy(src.at[pl.ds(base, C)], idx_refs[b][j],
                                      sem_i.at[b]).wait()

        def out_slice(i):
            return out_h.at[pl.ds((base0 + i * C) * D, C * D)]

        start_idx(0, 0)
        start_idx(1, 1)

        def outer(g, carry):
            for b in range(2):
                i = 2 * g + b
                wait_idx(i, b)

                @pl.when(i >= 2)
                def _():
                    pltpu.make_async_copy(out_bufs[b], out_slice(i - 2),
                                          sem_o.at[b]).wait()

                ob = out_bufs[b]
                ic, ih, isp = idx_refs[b]

                def group_body(g2, c2):
                    t0 = g2 * 16
                    sl = pl.ds(t0, 16)
                    comb = (ic[sl] | lax.shift_left(ih[sl], 4)
                            | lax.shift_left(isp[sl], 12))
                    for l in range(16):
                        x = comb[l]
                        ac = lax.shift_left(x & 15, 5)
                        ah = lax.shift_left(lax.shift_right_logical(x, 4)
                                            & 255, 5)
                        asp = lax.shift_left(lax.shift_right_logical(x, 12), 5)
                        o = (t0 + l) * D
                        for half in range(2):
                            hw = 16 * half
                            s = (plsc.bitcast(tabc[pl.ds(ac + hw, 16)],
                                              jnp.bfloat16)
                                 + plsc.bitcast(tabh[pl.ds(ah + hw, 16)],
                                                jnp.bfloat16)
                                 + plsc.bitcast(tabs[pl.ds(asp + hw, 16)],
                                                jnp.bfloat16))
                            su = plsc.bitcast(s, jnp.int32)
                            lo = plsc.bitcast(lax.shift_left(su, 16),
                                              jnp.float32)
                            hi = plsc.bitcast(su & mask_hi, jnp.float32)
                            ob[pl.ds(o + 32 * half, 16)] = lo
                            ob[pl.ds(o + 32 * half + 16, 16)] = hi
                    return c2

                lax.fori_loop(0, C // 16, group_body, 0)
                pltpu.async_copy(ob, out_slice(i), sem_o.at[b])

                @pl.when(i + 2 < nch)
                def _():
                    start_idx(i + 2, b)
            return carry

        lax.fori_loop(0, nch // 2, outer, 0)
        for b in range(2):
            pltpu.make_async_copy(out_bufs[b], out_slice(nch - 2 + b),
                                  sem_o.at[b]).wait()

    return k(colors, chars, specials, pc, ph, ps)


def kernel(colors, chars, specials, emb_colors, emb_chars, emb_specials,
           lin_w, lin_b):
    B, H, W = colors.shape
    N = B * H * W
    pc, ph, ps = _fold_tables(emb_colors, emb_chars, emb_specials, lin_w,
                              lin_b.reshape(1, D))
    out_flat = _sc_embed(
        colors.reshape(N), chars.reshape(N), specials.reshape(N),
        _pack_pairs(pc), _pack_pairs(ph), _pack_pairs(ps))
    return out_flat.reshape(B, H, W, D)


# speedup vs baseline: 3.1252x; 1.0034x over previous
import functools

import jax
import jax.numpy as jnp
from jax import lax
from jax.experimental import pallas as pl
from jax.experimental.pallas import tpu as pltpu
from jax.experimental.pallas import tpu_sc as plsc

D = 64
DW = D // 2
NC = 2
NS = 16
NW = NC * NS
C = 112


def _fold_tables(emb_colors, emb_chars, emb_specials, lin_w, lin_b2d):
    def body(ec, eh, es, w, bvec, pc, ph, ps):
        pc[...] = (jnp.dot(ec[...], w[0:D, :],
                           preferred_element_type=jnp.float32)
                   + bvec[...]).astype(jnp.bfloat16)
        ph[...] = jnp.dot(eh[...], w[D:2 * D, :],
                          preferred_element_type=jnp.float32).astype(jnp.bfloat16)
        ps[...] = jnp.dot(es[...], w[2 * D:3 * D, :],
                          preferred_element_type=jnp.float32).astype(jnp.bfloat16)

    return pl.pallas_call(
        body,
        out_shape=(
            jax.ShapeDtypeStruct((16, D), jnp.bfloat16),
            jax.ShapeDtypeStruct((256, D), jnp.bfloat16),
            jax.ShapeDtypeStruct((256, D), jnp.bfloat16),
        ),
    )(emb_colors, emb_chars, emb_specials, lin_w, lin_b2d)


def _pack_pairs(t):
    v = t.shape[0]
    tp = t.reshape(v, 2, 2, 16).transpose(0, 1, 3, 2)
    return lax.bitcast_convert_type(tp.reshape(v, DW, 2),
                                    jnp.int32).reshape(v * DW)


def _sc_embed(colors, chars, specials, pc, ph, ps):
    N = colors.shape[0]
    assert N % (NW * C) == 0
    tpw = N // NW
    nch = tpw // C
    assert nch % 2 == 0

    mesh = plsc.VectorSubcoreMesh(core_axis_name="c", subcore_axis_name="s")

    @functools.partial(
        pl.kernel,
        out_type=jax.ShapeDtypeStruct((N * D,), jnp.float32),
        mesh=mesh,
        compiler_params=pltpu.CompilerParams(needs_layout_passes=False),
        scratch_types=[
            pltpu.VMEM((16 * DW,), jnp.int32),
            pltpu.VMEM((256 * DW,), jnp.int32),
            pltpu.VMEM((256 * DW,), jnp.int32),
            pltpu.VMEM((C,), jnp.int32),
            pltpu.VMEM((C,), jnp.int32),
            pltpu.VMEM((C,), jnp.int32),
            pltpu.VMEM((C,), jnp.int32),
            pltpu.VMEM((C,), jnp.int32),
            pltpu.VMEM((C,), jnp.int32),
            pltpu.VMEM((C * D,), jnp.float32),
            pltpu.VMEM((C * D,), jnp.float32),
            pltpu.SemaphoreType.DMA((2,)),
            pltpu.SemaphoreType.DMA((2,)),
        ],
    )
    def k(colors_h, chars_h, specials_h, pc_h, ph_h, ps_h, out_h,
          tabc, tabh, tabs, ic0, ih0, is0, ic1, ih1, is1, ob0, ob1,
          sem_i, sem_o):
        idx_refs = ((ic0, ih0, is0), (ic1, ih1, is1))
        out_bufs = (ob0, ob1)
        wid = lax.axis_index("s") * NC + lax.axis_index("c")
        base0 = wid * tpw

        pltpu.sync_copy(pc_h, tabc)
        pltpu.sync_copy(ph_h, tabh)
        pltpu.sync_copy(ps_h, tabs)

        mask_hi = jnp.full((16,), -65536, dtype=jnp.int32)

        idx_srcs = (colors_h, chars_h, specials_h)

        def start_idx(i, b):
            base = base0 + i * C
            for j, src in enumerate(idx_srcs):
                pltpu.async_copy(src.at[pl.ds(base, C)], idx_refs[b][j],
                                 sem_i.at[b])

        def wait_idx(i, b):
            base = base0 + i * C
            for j, src in enumerate(idx_srcs):
                pltpu.make_async_copy(src.at[pl.ds(base, C)], idx_refs[b][j],
                                      sem_i.at[b]).wait()

        def out_slice(i):
            return out_h.at[pl.ds((base0 + i * C) * D, C * D)]

        start_idx(0, 0)
        start_idx(1, 1)

        def outer(g, carry):
            for b in range(2):
                i = 2 * g + b
                wait_idx(i, b)

                @pl.when(i >= 2)
                def _():
                    pltpu.make_async_copy(out_bufs[b], out_slice(i - 2),
                                          sem_o.at[b]).wait()

                ob = out_bufs[b]
                ic, ih, isp = idx_refs[b]

                @plsc.parallel_loop(0, C // 16, unroll=2)
                def group_body(g2):
                    t0 = g2 * 16
                    sl = pl.ds(t0, 16)
                    comb = (ic[sl] | lax.shift_left(ih[sl], 4)
                            | lax.shift_left(isp[sl], 12))
                    xs = [comb[l] for l in range(16)]
                    acs = [lax.shift_left(x & 15, 5) for x in xs]
                    ahs = [lax.shift_left(x & 4080, 1) for x in xs]
                    asps = [lax.shift_right_logical(x & 1044480, 7)
                            for x in xs]
                    for l in range(16):
                        o = (t0 + l) * D
                        for half in range(2):
                            hw = 16 * half
                            s = (plsc.bitcast(tabc[pl.ds(acs[l] + hw, 16)],
                                              jnp.bfloat16)
                                 + plsc.bitcast(tabh[pl.ds(ahs[l] + hw, 16)],
                                                jnp.bfloat16)
                                 + plsc.bitcast(tabs[pl.ds(asps[l] + hw, 16)],
                                                jnp.bfloat16))
                            su = plsc.bitcast(s, jnp.int32)
                            lo = plsc.bitcast(lax.shift_left(su, 16),
                                              jnp.float32)
                            hi = plsc.bitcast(su & mask_hi, jnp.float32)
                            ob[pl.ds(o + 32 * half, 16)] = lo
                            ob[pl.ds(o + 32 * half + 16, 16)] = hi
                pltpu.async_copy(ob, out_slice(i), sem_o.at[b])

                @pl.when(i + 2 < nch)
                def _():
                    start_idx(i + 2, b)
            return carry

        lax.fori_loop(0, nch // 2, outer, 0)
        for b in range(2):
            pltpu.make_async_copy(out_bufs[b], out_slice(nch - 2 + b),
                                  sem_o.at[b]).wait()

    return k(colors, chars, specials, pc, ph, ps)


def kernel(colors, chars, specials, emb_colors, emb_chars, emb_specials,
           lin_w, lin_b):
    B, H, W = colors.shape
    N = B * H * W
    pc, ph, ps = _fold_tables(emb_colors, emb_chars, emb_specials, lin_w,
                              lin_b.reshape(1, D))
    out_flat = _sc_embed(
        colors.reshape(N), chars.reshape(N), specials.reshape(N),
        _pack_pairs(pc), _pack_pairs(ph), _pack_pairs(ps))
    return out_flat.reshape(B, H, W, D)


# speedup vs baseline: 3.8574x; 1.2343x over previous
import functools

import jax
import jax.numpy as jnp
from jax import lax
from jax.experimental import pallas as pl
from jax.experimental.pallas import tpu as pltpu
from jax.experimental.pallas import tpu_sc as plsc

D = 64
DW = D // 2
NC = 2
NS = 16
NW = NC * NS
C = 336


def _fold_tables(emb_colors, emb_chars, emb_specials, lin_w, lin_b2d):
    def body(ec, eh, es, w, bvec, pc, ph, ps):
        pc[...] = (jnp.dot(ec[...], w[0:D, :],
                           preferred_element_type=jnp.float32)
                   + bvec[...]).astype(jnp.bfloat16)
        ph[...] = jnp.dot(eh[...], w[D:2 * D, :],
                          preferred_element_type=jnp.float32).astype(jnp.bfloat16)
        ps[...] = jnp.dot(es[...], w[2 * D:3 * D, :],
                          preferred_element_type=jnp.float32).astype(jnp.bfloat16)

    return pl.pallas_call(
        body,
        out_shape=(
            jax.ShapeDtypeStruct((16, D), jnp.bfloat16),
            jax.ShapeDtypeStruct((256, D), jnp.bfloat16),
            jax.ShapeDtypeStruct((256, D), jnp.bfloat16),
        ),
    )(emb_colors, emb_chars, emb_specials, lin_w, lin_b2d)


def _pack_pairs(t):
    v = t.shape[0]
    tp = t.reshape(v, 2, 2, 16).transpose(0, 1, 3, 2)
    return lax.bitcast_convert_type(tp.reshape(v, DW, 2),
                                    jnp.int32).reshape(v * DW)


def _sc_embed(colors, chars, specials, pc, ph, ps):
    N = colors.shape[0]
    assert N % (NW * C) == 0
    tpw = N // NW
    nch = tpw // C
    assert nch % 2 == 0

    mesh = plsc.VectorSubcoreMesh(core_axis_name="c", subcore_axis_name="s")

    @functools.partial(
        pl.kernel,
        out_type=jax.ShapeDtypeStruct((N * D,), jnp.float32),
        mesh=mesh,
        compiler_params=pltpu.CompilerParams(needs_layout_passes=False),
        scratch_types=[
            pltpu.VMEM((16 * DW,), jnp.int32),
            pltpu.VMEM((256 * DW,), jnp.int32),
            pltpu.VMEM((256 * DW,), jnp.int32),
            pltpu.VMEM((C,), jnp.int32),
            pltpu.VMEM((C,), jnp.int32),
            pltpu.VMEM((C,), jnp.int32),
            pltpu.VMEM((C,), jnp.int32),
            pltpu.VMEM((C,), jnp.int32),
            pltpu.VMEM((C,), jnp.int32),
            pltpu.VMEM((C * D,), jnp.float32),
            pltpu.VMEM((C * D,), jnp.float32),
            pltpu.SemaphoreType.DMA((2,)),
            pltpu.SemaphoreType.DMA((2,)),
        ],
    )
    def k(colors_h, chars_h, specials_h, pc_h, ph_h, ps_h, out_h,
          tabc, tabh, tabs, ic0, ih0, is0, ic1, ih1, is1, ob0, ob1,
          sem_i, sem_o):
        idx_refs = ((ic0, ih0, is0), (ic1, ih1, is1))
        out_bufs = (ob0, ob1)
        wid = lax.axis_index("s") * NC + lax.axis_index("c")
        base0 = wid * tpw

        pltpu.sync_copy(pc_h, tabc)
        pltpu.sync_copy(ph_h, tabh)
        pltpu.sync_copy(ps_h, tabs)

        mask_hi = jnp.full((16,), -65536, dtype=jnp.int32)

        idx_srcs = (colors_h, chars_h, specials_h)

        def start_idx(i, b):
            base = base0 + i * C
            for j, src in enumerate(idx_srcs):
                pltpu.async_copy(src.at[pl.ds(base, C)], idx_refs[b][j],
                                 sem_i.at[b])

        def wait_idx(i, b):
            base = base0 + i * C
            for j, src in enumerate(idx_srcs):
                pltpu.make_async_copy(src.at[pl.ds(base, C)], idx_refs[b][j],
                                      sem_i.at[b]).wait()

        def out_slice(i):
            return out_h.at[pl.ds((base0 + i * C) * D, C * D)]

        start_idx(0, 0)
        start_idx(1, 1)

        def outer(g, carry):
            for b in range(2):
                i = 2 * g + b
                wait_idx(i, b)

                @pl.when(i >= 2)
                def _():
                    pltpu.make_async_copy(out_bufs[b], out_slice(i - 2),
                                          sem_o.at[b]).wait()

                ob = out_bufs[b]
                ic, ih, isp = idx_refs[b]

                @plsc.parallel_loop(0, C // 16, unroll=2)
                def group_body(g2):
                    t0 = g2 * 16
                    sl = pl.ds(t0, 16)
                    comb = (ic[sl] | lax.shift_left(ih[sl], 4)
                            | lax.shift_left(isp[sl], 12))
                    xs = [comb[l] for l in range(16)]
                    acs = [lax.shift_left(x & 15, 5) for x in xs]
                    ahs = [lax.shift_left(x & 4080, 1) for x in xs]
                    asps = [lax.shift_right_logical(x & 1044480, 7)
                            for x in xs]
                    for l in range(16):
                        o = (t0 + l) * D
                        for half in range(2):
                            hw = 16 * half
                            s = (plsc.bitcast(tabc[pl.ds(acs[l] + hw, 16)],
                                              jnp.bfloat16)
                                 + plsc.bitcast(tabh[pl.ds(ahs[l] + hw, 16)],
                                                jnp.bfloat16)
                                 + plsc.bitcast(tabs[pl.ds(asps[l] + hw, 16)],
                                                jnp.bfloat16))
                            su = plsc.bitcast(s, jnp.int32)
                            lo = plsc.bitcast(lax.shift_left(su, 16),
                                              jnp.float32)
                            hi = plsc.bitcast(su & mask_hi, jnp.float32)
                            ob[pl.ds(o + 32 * half, 16)] = lo
                            ob[pl.ds(o + 32 * half + 16, 16)] = hi
                pltpu.async_copy(ob, out_slice(i), sem_o.at[b])

                @pl.when(i + 2 < nch)
                def _():
                    start_idx(i + 2, b)
            return carry

        lax.fori_loop(0, nch // 2, outer, 0)
        for b in range(2):
            pltpu.make_async_copy(out_bufs[b], out_slice(nch - 2 + b),
                                  sem_o.at[b]).wait()

    return k(colors, chars, specials, pc, ph, ps)


def kernel(colors, chars, specials, emb_colors, emb_chars, emb_specials,
           lin_w, lin_b):
    B, H, W = colors.shape
    N = B * H * W
    pc, ph, ps = _fold_tables(emb_colors, emb_chars, emb_specials, lin_w,
                              lin_b.reshape(1, D))
    out_flat = _sc_embed(
        colors.reshape(N), chars.reshape(N), specials.reshape(N),
        _pack_pairs(pc), _pack_pairs(ph), _pack_pairs(ps))
    return out_flat.reshape(B, H, W, D)


# speedup vs baseline: 3.8631x; 1.0015x over previous
import functools

import jax
import jax.numpy as jnp
from jax import lax
from jax.experimental import pallas as pl
from jax.experimental.pallas import tpu as pltpu
from jax.experimental.pallas import tpu_sc as plsc

D = 64
DW = D // 2
NC = 2
NS = 16
NW = NC * NS
WR = 79
WP = 80
GPC = 4
C = GPC * WP


def _fold_tables(emb_colors, emb_chars, emb_specials, lin_w, lin_b2d):
    def body(ec, eh, es, w, bvec, pc, ph, ps):
        pc[...] = (jnp.dot(ec[...], w[0:D, :],
                           preferred_element_type=jnp.float32)
                   + bvec[...]).astype(jnp.bfloat16)
        ph[...] = jnp.dot(eh[...], w[D:2 * D, :],
                          preferred_element_type=jnp.float32).astype(jnp.bfloat16)
        ps[...] = jnp.dot(es[...], w[2 * D:3 * D, :],
                          preferred_element_type=jnp.float32).astype(jnp.bfloat16)

    return pl.pallas_call(
        body,
        out_shape=(
            jax.ShapeDtypeStruct((16, D), jnp.bfloat16),
            jax.ShapeDtypeStruct((256, D), jnp.bfloat16),
            jax.ShapeDtypeStruct((256, D), jnp.bfloat16),
        ),
    )(emb_colors, emb_chars, emb_specials, lin_w, lin_b2d)


def _pack_pairs(t):
    v = t.shape[0]
    tp = t.reshape(v, 2, 2, 16).transpose(0, 1, 3, 2)
    return lax.bitcast_convert_type(tp.reshape(v, DW, 2),
                                    jnp.int32).reshape(v * DW)


def _sc_embed(colors, chars, specials, pc, ph, ps, B, H):
    BH = B * H
    assert BH % (NW * GPC) == 0
    rpw = BH // NW
    nch = rpw // GPC
    assert nch % 2 == 0

    mesh = plsc.VectorSubcoreMesh(core_axis_name="c", subcore_axis_name="s")

    @functools.partial(
        pl.kernel,
        out_type=jax.ShapeDtypeStruct((B, H, WR, D), jnp.float32),
        mesh=mesh,
        compiler_params=pltpu.CompilerParams(needs_layout_passes=False),
        scratch_types=[
            pltpu.VMEM((16 * DW,), jnp.int32),
            pltpu.VMEM((256 * DW,), jnp.int32),
            pltpu.VMEM((256 * DW,), jnp.int32),
            pltpu.VMEM((C,), jnp.int32),
            pltpu.VMEM((C,), jnp.int32),
            pltpu.VMEM((C,), jnp.int32),
            pltpu.VMEM((C,), jnp.int32),
            pltpu.VMEM((C,), jnp.int32),
            pltpu.VMEM((C,), jnp.int32),
            pltpu.VMEM((WP, D), jnp.float32),
            pltpu.VMEM((WP, D), jnp.float32),
            pltpu.VMEM((WP, D), jnp.float32),
            pltpu.VMEM((WP, D), jnp.float32),
            pltpu.VMEM((WP, D), jnp.float32),
            pltpu.VMEM((WP, D), jnp.float32),
            pltpu.VMEM((WP, D), jnp.float32),
            pltpu.VMEM((WP, D), jnp.float32),
            pltpu.SemaphoreType.DMA((2,)),
            pltpu.SemaphoreType.DMA((2,)),
        ],
    )
    def k(colors_h, chars_h, specials_h, pc_h, ph_h, ps_h, out_h,
          tabc, tabh, tabs, ic0, ih0, is0, ic1, ih1, is1,
          oa0, oa1, oa2, oa3, ob0, ob1, ob2, ob3,
          sem_i, sem_o):
        idx_refs = ((ic0, ih0, is0), (ic1, ih1, is1))
        out_bufs = ((oa0, oa1, oa2, oa3), (ob0, ob1, ob2, ob3))
        wid = lax.axis_index("s") * NC + lax.axis_index("c")
        row0 = wid * rpw

        pltpu.sync_copy(pc_h, tabc)
        pltpu.sync_copy(ph_h, tabh)
        pltpu.sync_copy(ps_h, tabs)

        mask_hi = jnp.full((16,), -65536, dtype=jnp.int32)

        idx_srcs = (colors_h, chars_h, specials_h)

        def start_idx(i, b):
            base = (row0 + i * GPC) * WP
            for j, src in enumerate(idx_srcs):
                pltpu.async_copy(src.at[pl.ds(base, C)], idx_refs[b][j],
                                 sem_i.at[b])

        def wait_idx(i, b):
            base = (row0 + i * GPC) * WP
            for j, src in enumerate(idx_srcs):
                pltpu.make_async_copy(src.at[pl.ds(base, C)], idx_refs[b][j],
                                      sem_i.at[b]).wait()

        def out_dst(i, gi):
            bh = row0 + i * GPC + gi
            return out_h.at[bh // H, bh % H]

        start_idx(0, 0)
        start_idx(1, 1)

        def outer(g, carry):
            for b in range(2):
                i = 2 * g + b
                wait_idx(i, b)

                @pl.when(i >= 2)
                def _():
                    for gi in range(GPC):
                        pltpu.make_async_copy(
                            out_bufs[b][gi].at[pl.ds(0, WR)],
                            out_dst(i - 2, gi), sem_o.at[b]).wait()

                ic, ih, isp = idx_refs[b]

                for gi in range(GPC):
                    obg = out_bufs[b][gi]

                    @plsc.parallel_loop(0, WP // 16)
                    def group_body(g2):
                        t0 = g2 * 16
                        sl = pl.ds(gi * WP + t0, 16)
                        comb = (ic[sl] | lax.shift_left(ih[sl], 4)
                                | lax.shift_left(isp[sl], 12))
                        xs = [comb[l] for l in range(16)]
                        acs = [lax.shift_left(x & 15, 5) for x in xs]
                        ahs = [lax.shift_left(x & 4080, 1) for x in xs]
                        asps = [lax.shift_right_logical(x & 1044480, 7)
                                for x in xs]
                        for l in range(16):
                            for half in range(2):
                                hw = 16 * half
                                s = (plsc.bitcast(
                                        tabc[pl.ds(acs[l] + hw, 16)],
                                        jnp.bfloat16)
                                     + plsc.bitcast(
                                         tabh[pl.ds(ahs[l] + hw, 16)],
                                         jnp.bfloat16)
                                     + plsc.bitcast(
                                         tabs[pl.ds(asps[l] + hw, 16)],
                                         jnp.bfloat16))
                                su = plsc.bitcast(s, jnp.int32)
                                lo = plsc.bitcast(lax.shift_left(su, 16),
                                                  jnp.float32)
                                hi = plsc.bitcast(su & mask_hi, jnp.float32)
                                obg[t0 + l, pl.ds(32 * half, 16)] = lo
                                obg[t0 + l, pl.ds(32 * half + 16, 16)] = hi

                    pltpu.async_copy(obg.at[pl.ds(0, WR)], out_dst(i, gi),
                                     sem_o.at[b])

                @pl.when(i + 2 < nch)
                def _():
                    start_idx(i + 2, b)
            return carry

        lax.fori_loop(0, nch // 2, outer, 0)
        for b in range(2):
            for gi in range(GPC):
                pltpu.make_async_copy(out_bufs[b][gi].at[pl.ds(0, WR)],
                                      out_dst(nch - 2 + b, gi),
                                      sem_o.at[b]).wait()

    return k(colors, chars, specials, pc, ph, ps)


def kernel(colors, chars, specials, emb_colors, emb_chars, emb_specials,
           lin_w, lin_b):
    B, H, W = colors.shape
    BH = B * H

    def padw(a):
        return jnp.pad(a.reshape(BH, W), ((0, 0), (0, WP - W))).reshape(-1)

    pc, ph, ps = _fold_tables(emb_colors, emb_chars, emb_specials, lin_w,
                              lin_b.reshape(1, D))
    return _sc_embed(padw(colors), padw(chars), padw(specials),
                     _pack_pairs(pc), _pack_pairs(ph), _pack_pairs(ps), B, H)
